# baseline (speedup 1.0000x reference)
import functools

import jax
import jax.numpy as jnp
from jax import lax
from jax.experimental import pallas as pl
from jax.experimental.pallas import tpu as pltpu
from jax.experimental.pallas import tpu_sc as plsc

_SC_PARAMS = pltpu.CompilerParams(needs_layout_passes=False)

E = 8
T = 128
NC, NS, L = 2, 16, 16
NW = NC * NS


def _router_body(x_ref, wr_ref, br_ref, eid_ref):
    xv = x_ref[0]
    logits = jnp.dot(xv, wr_ref[...], preferred_element_type=jnp.float32)
    logits = logits + br_ref[...]
    maxv = jnp.max(logits, axis=-1, keepdims=True)
    li = lax.broadcasted_iota(jnp.int32, logits.shape, 1)
    cand = jnp.where(logits == maxv, li, jnp.int32(1 << 30))
    eid_ref[...] = jnp.min(cand, axis=-1)


def _make_count_body(s):
    tp = s // NW
    nch = tp // L

    def _body(eid_hbm, cnts_hbm, eid_l, cnt_v):
        wid = lax.axis_index("s") * NC + lax.axis_index("c")
        pltpu.sync_copy(eid_hbm.at[pl.ds(wid * tp, tp)], eid_l)
        lane = lax.iota(jnp.int32, L)
        cnt = jnp.zeros((L,), jnp.int32)
        for c in range(nch):
            v = eid_l[pl.ds(c * L, L)]
            for e in range(E):
                se = jnp.sum((v == e).astype(jnp.int32))
                cnt = jnp.where(lane == e, cnt + se, cnt)
        cnt_v[...] = cnt
        pltpu.sync_copy(cnt_v, cnts_hbm.at[wid])

    return _body


def _make_dispatch_body(s, P, Upad):
    tp = s // NW
    nch = tp // L

    def _body(eid_hbm, x_hbm, cnts_hbm, pos_hbm, xs_hbm, te_hbm, vt_hbm,
              ft_hbm, cnts_l, eid_l, pos_v, xrows_v, te_v, vt_v, ft_v, sem):
        wid = lax.axis_index("s") * NC + lax.axis_index("c")
        pltpu.sync_copy(cnts_hbm, cnts_l)
        pltpu.sync_copy(eid_hbm.at[pl.ds(wid * tp, tp)], eid_l)

        tot = jnp.zeros((L,), jnp.int32)
        pre = jnp.zeros((L,), jnp.int32)
        for w in range(NW):
            row = cnts_l[w]
            tot = tot + row
            before = jnp.full((L,), w, jnp.int32) < wid
            pre = jnp.where(before, pre + row, pre)

        tiles_vec = (tot + (T - 1)) // T
        cumt = plsc.cumsum(tiles_vec)
        base_vec = (cumt - tiles_vec) * T
        start = base_vec + pre

        offs = [start[e] for e in range(E)]

        for c in range(nch):
            v = eid_l[pl.ds(c * L, L)]
            posv = jnp.zeros((L,), jnp.int32)
            for e in range(E):
                m = v == e
                mi = m.astype(jnp.int32)
                csum = plsc.cumsum(mi)
                posv = jnp.where(m, offs[e] + csum - 1, posv)
                offs[e] = offs[e] + jnp.sum(mi)
            pos_v[pl.ds(c * L, L)] = posv

        pltpu.sync_copy(pos_v, pos_hbm.at[pl.ds(wid * tp, tp)])

        pltpu.sync_copy(x_hbm.at[0, pl.ds(wid * tp, tp)], xrows_v)
        pltpu.async_copy(xrows_v, xs_hbm.at[pos_v], sem).wait()

        @pl.when(wid == 0)
        def _():
            nt = cumt[E - 1]
            lane = lax.iota(jnp.int32, L)
            last_e = jnp.max(jnp.where(tiles_vec > 0, lane, 0))
            cum_s = [cumt[e] for e in range(E)]
            tb_s = [cumt[e] - tiles_vec[e] for e in range(E)]
            for c in range(Upad // L):
                u = lane + c * L
                acc = jnp.zeros((L,), jnp.int32)
                ft = jnp.zeros((L,), jnp.int32)
                for e in range(E):
                    acc = acc + (u >= cum_s[e]).astype(jnp.int32)
                    ft = ft + (u == tb_s[e]).astype(jnp.int32)
                te_v[pl.ds(c * L, L)] = jnp.where(acc > E - 1, last_e, acc)
                vt_v[pl.ds(c * L, L)] = (u < nt).astype(jnp.int32)
                ft_v[pl.ds(c * L, L)] = jnp.minimum(ft, 1)
            pltpu.sync_copy(te_v, te_hbm)
            pltpu.sync_copy(vt_v, vt_hbm)
            pltpu.sync_copy(ft_v, ft_hbm)

    return _body


def _make_combine_body(s):
    rp = s // NW

    def _body(tab_hbm, idx_hbm, out_hbm, idx_v, rows_v, sem):
        wid = lax.axis_index("s") * NC + lax.axis_index("c")
        base = wid * rp
        pltpu.sync_copy(idx_hbm.at[pl.ds(base, rp)], idx_v)
        pltpu.async_copy(tab_hbm.at[idx_v], rows_v, sem).wait()
        pltpu.sync_copy(rows_v, out_hbm.at[0, pl.ds(base, rp)])

    return _body


def _ffn_body(te_ref, vt_ref, ft_ref, xs_ref, w1_ref, b1_ref, w2_ref, b2_ref,
              ys_ref, w1b_ref, w2b_ref):
    u = pl.program_id(0)

    @pl.when(vt_ref[u] == 1)
    def _():
        e = te_ref[u]

        @pl.when(ft_ref[u] == 1)
        def _():
            w1b_ref[...] = w1_ref[0].astype(jnp.bfloat16)
            w2b_ref[...] = w2_ref[0].astype(jnp.bfloat16)

        xb = xs_ref[...].astype(jnp.bfloat16)
        hpre = jnp.dot(xb, w1b_ref[...], preferred_element_type=jnp.float32)
        hpre = hpre + b1_ref[pl.ds(e, 1), :]
        hv = 0.5 * hpre * (1.0 + lax.erf(hpre * 0.7071067811865476))
        hb = hv.astype(jnp.bfloat16)
        yv = jnp.dot(hb, w2b_ref[...], preferred_element_type=jnp.float32)
        ys_ref[...] = yv + b2_ref[pl.ds(e, 1), :]


def kernel(x, Wr, br, W1, b1, W2, b2):
    b, s, d = x.shape
    e_, _, h = W1.shape

    U = s // T + E
    Upad = ((U + L - 1) // L) * L
    P = U * T

    wr_p = jnp.pad(Wr, ((0, 0), (0, 128 - E)))
    br_p = jnp.concatenate([br, jnp.full((128 - E,), -1e30, jnp.float32)])[None, :]
    eid = pl.pallas_call(
        _router_body,
        out_shape=jax.ShapeDtypeStruct((s,), jnp.int32),
    )(x, wr_p, br_p)

    mesh = plsc.VectorSubcoreMesh(core_axis_name="c", subcore_axis_name="s")

    cnts = pl.kernel(
        _make_count_body(s),
        out_type=jax.ShapeDtypeStruct((NW, L), jnp.int32),
        mesh=mesh,
        scratch_types=[pltpu.VMEM((s // NW,), jnp.int32),
                       pltpu.VMEM((L,), jnp.int32)],
        compiler_params=_SC_PARAMS,
    )(eid)

    pos, xs, te, vt, ft = pl.kernel(
        _make_dispatch_body(s, P, Upad),
        out_type=[jax.ShapeDtypeStruct((s,), jnp.int32),
                  jax.ShapeDtypeStruct((P, d), jnp.float32),
                  jax.ShapeDtypeStruct((Upad,), jnp.int32),
                  jax.ShapeDtypeStruct((Upad,), jnp.int32),
                  jax.ShapeDtypeStruct((Upad,), jnp.int32)],
        mesh=mesh,
        scratch_types=[pltpu.VMEM((NW, L), jnp.int32),
                       pltpu.VMEM((s // NW,), jnp.int32),
                       pltpu.VMEM((s // NW,), jnp.int32),
                       pltpu.VMEM((s // NW, d), jnp.float32),
                       pltpu.VMEM((Upad,), jnp.int32),
                       pltpu.VMEM((Upad,), jnp.int32),
                       pltpu.VMEM((Upad,), jnp.int32),
                       pltpu.SemaphoreType.DMA],
        compiler_params=_SC_PARAMS,
    )(eid, x, cnts)

    grid_spec = pltpu.PrefetchScalarGridSpec(
        num_scalar_prefetch=3,
        grid=(U,),
        in_specs=[
            pl.BlockSpec((T, d), lambda u, te_r, vt_r, ft_r: (u, 0)),
            pl.BlockSpec((1, d, h), lambda u, te_r, vt_r, ft_r: (te_r[u], 0, 0)),
            pl.BlockSpec((E, h), lambda u, te_r, vt_r, ft_r: (0, 0)),
            pl.BlockSpec((1, h, d), lambda u, te_r, vt_r, ft_r: (te_r[u], 0, 0)),
            pl.BlockSpec((E, d), lambda u, te_r, vt_r, ft_r: (0, 0)),
        ],
        out_specs=pl.BlockSpec((T, d), lambda u, te_r, vt_r, ft_r: (u, 0)),
        scratch_shapes=[pltpu.VMEM((d, h), jnp.bfloat16),
                        pltpu.VMEM((h, d), jnp.bfloat16)],
    )
    ys = pl.pallas_call(
        _ffn_body,
        grid_spec=grid_spec,
        out_shape=jax.ShapeDtypeStruct((P, d), jnp.float32),
    )(te, vt, ft, xs, W1, b1, W2, b2)

    out = pl.kernel(
        _make_combine_body(s),
        out_type=jax.ShapeDtypeStruct((b, s, d), jnp.float32),
        mesh=mesh,
        scratch_types=[pltpu.VMEM((s // NW,), jnp.int32),
                       pltpu.VMEM((s // NW, d), jnp.float32),
                       pltpu.SemaphoreType.DMA],
        compiler_params=_SC_PARAMS,
    )(ys, pos)

    return out


# speedup vs baseline: 1.1038x; 1.1038x over previous
import functools

import jax
import jax.numpy as jnp
from jax import lax
from jax.experimental import pallas as pl
from jax.experimental.pallas import tpu as pltpu
from jax.experimental.pallas import tpu_sc as plsc

_SC_PARAMS = pltpu.CompilerParams(needs_layout_passes=False)

E = 8
T = 128
NC, NS, L = 2, 16, 16
NW = NC * NS


def _make_router_body(s):
    tp = s // NW

    def _router_body(x_ref, wr_ref, br_ref, eid_ref, cnts_ref):
        xv = x_ref[0]
        logits = jnp.dot(xv, wr_ref[...], preferred_element_type=jnp.float32)
        logits = logits + br_ref[...]
        maxv = jnp.max(logits, axis=-1, keepdims=True)
        li = lax.broadcasted_iota(jnp.int32, logits.shape, 1)
        cand = jnp.where(logits == maxv, li, jnp.int32(1 << 30))
        eid = jnp.min(cand, axis=-1)
        eid_ref[...] = eid
        onehot = (li == eid[:, None]).astype(jnp.float32)
        aw = lax.broadcasted_iota(jnp.int32, (NW, s), 0)
        at = lax.broadcasted_iota(jnp.int32, (NW, s), 1) // tp
        amat = (aw == at).astype(jnp.float32)
        cnts_f = jnp.dot(amat, onehot, preferred_element_type=jnp.float32)
        cnts_ref[...] = lax.slice(cnts_f, (0, 0), (NW, L)).astype(jnp.int32)

    return _router_body


def _make_dispatch_body(s, P, Upad):
    tp = s // NW
    nch = tp // L

    def _body(eid_hbm, x_hbm, cnts_hbm, pos_hbm, xs_hbm, te_hbm, vt_hbm,
              cnts_l, eid_l, pos_v, xrows_v, te_v, vt_v, sem):
        wid = lax.axis_index("s") * NC + lax.axis_index("c")
        pltpu.sync_copy(cnts_hbm, cnts_l)
        pltpu.sync_copy(eid_hbm.at[pl.ds(wid * tp, tp)], eid_l)

        tot = jnp.zeros((L,), jnp.int32)
        pre = jnp.zeros((L,), jnp.int32)
        for w in range(NW):
            row = cnts_l[w]
            tot = tot + row
            before = jnp.full((L,), w, jnp.int32) < wid
            pre = jnp.where(before, pre + row, pre)

        tiles_vec = (tot + (T - 1)) // T
        cumt = plsc.cumsum(tiles_vec)
        base_vec = (cumt - tiles_vec) * T
        start = base_vec + pre

        offs = [start[e] for e in range(E)]

        for c in range(nch):
            v = eid_l[pl.ds(c * L, L)]
            posv = jnp.zeros((L,), jnp.int32)
            for e in range(E):
                m = v == e
                mi = m.astype(jnp.int32)
                csum = plsc.cumsum(mi)
                posv = jnp.where(m, offs[e] + csum - 1, posv)
                offs[e] = offs[e] + jnp.sum(mi)
            pos_v[pl.ds(c * L, L)] = posv

        pltpu.sync_copy(pos_v, pos_hbm.at[pl.ds(wid * tp, tp)])

        pltpu.sync_copy(x_hbm.at[0, pl.ds(wid * tp, tp)], xrows_v)
        pltpu.async_copy(xrows_v, xs_hbm.at[pos_v], sem).wait()

        @pl.when(wid == 0)
        def _():
            nt = cumt[E - 1]
            lane = lax.iota(jnp.int32, L)
            last_e = jnp.max(jnp.where(tiles_vec > 0, lane, 0))
            cum_s = [cumt[e] for e in range(E)]
            for c in range(Upad // L):
                u = lane + c * L
                acc = jnp.zeros((L,), jnp.int32)
                for e in range(E):
                    acc = acc + (u >= cum_s[e]).astype(jnp.int32)
                te_v[pl.ds(c * L, L)] = jnp.where(acc > E - 1, last_e, acc)
                vt_v[pl.ds(c * L, L)] = (u < nt).astype(jnp.int32)
            pltpu.sync_copy(te_v, te_hbm)
            pltpu.sync_copy(vt_v, vt_hbm)

    return _body


def _make_combine_body(s):
    rp = s // NW

    def _body(tab_hbm, idx_hbm, out_hbm, idx_v, rows_v, sem):
        wid = lax.axis_index("s") * NC + lax.axis_index("c")
        base = wid * rp
        pltpu.sync_copy(idx_hbm.at[pl.ds(base, rp)], idx_v)
        pltpu.async_copy(tab_hbm.at[idx_v], rows_v, sem).wait()
        pltpu.sync_copy(rows_v, out_hbm.at[0, pl.ds(base, rp)])

    return _body


def _ffn_body(te_ref, vt_ref, xs_ref, w1_ref, b1_ref, w2_ref, b2_ref, ys_ref):
    u = pl.program_id(0)

    @pl.when(vt_ref[u] == 1)
    def _():
        e = te_ref[u]
        xv = xs_ref[...]
        hpre = jnp.dot(xv, w1_ref[0], preferred_element_type=jnp.float32)
        hpre = hpre + b1_ref[pl.ds(e, 1), :]
        hv = 0.5 * hpre * (1.0 + lax.erf(hpre * 0.7071067811865476))
        yv = jnp.dot(hv, w2_ref[0], preferred_element_type=jnp.float32)
        ys_ref[...] = yv + b2_ref[pl.ds(e, 1), :]


def kernel(x, Wr, br, W1, b1, W2, b2):
    b, s, d = x.shape
    e_, _, h = W1.shape

    U = s // T + E
    Upad = ((U + L - 1) // L) * L
    P = U * T

    wr_p = jnp.pad(Wr, ((0, 0), (0, 128 - E)))
    br_p = jnp.concatenate([br, jnp.full((128 - E,), -1e30, jnp.float32)])[None, :]
    eid, cnts = pl.pallas_call(
        _make_router_body(s),
        out_shape=[jax.ShapeDtypeStruct((s,), jnp.int32),
                   jax.ShapeDtypeStruct((NW, L), jnp.int32)],
    )(x, wr_p, br_p)

    mesh = plsc.VectorSubcoreMesh(core_axis_name="c", subcore_axis_name="s")

    pos, xs, te, vt = pl.kernel(
        _make_dispatch_body(s, P, Upad),
        out_type=[jax.ShapeDtypeStruct((s,), jnp.int32),
                  jax.ShapeDtypeStruct((P, d), jnp.float32),
                  jax.ShapeDtypeStruct((Upad,), jnp.int32),
                  jax.ShapeDtypeStruct((Upad,), jnp.int32)],
        mesh=mesh,
        scratch_types=[pltpu.VMEM((NW, L), jnp.int32),
                       pltpu.VMEM((s // NW,), jnp.int32),
                       pltpu.VMEM((s // NW,), jnp.int32),
                       pltpu.VMEM((s // NW, d), jnp.float32),
                       pltpu.VMEM((Upad,), jnp.int32),
                       pltpu.VMEM((Upad,), jnp.int32),
                       pltpu.SemaphoreType.DMA],
        compiler_params=_SC_PARAMS,
    )(eid, x, cnts)

    grid_spec = pltpu.PrefetchScalarGridSpec(
        num_scalar_prefetch=2,
        grid=(U,),
        in_specs=[
            pl.BlockSpec((T, d), lambda u, te_r, vt_r: (u, 0)),
            pl.BlockSpec((1, d, h), lambda u, te_r, vt_r: (te_r[u], 0, 0)),
            pl.BlockSpec((E, h), lambda u, te_r, vt_r: (0, 0)),
            pl.BlockSpec((1, h, d), lambda u, te_r, vt_r: (te_r[u], 0, 0)),
            pl.BlockSpec((E, d), lambda u, te_r, vt_r: (0, 0)),
        ],
        out_specs=pl.BlockSpec((T, d), lambda u, te_r, vt_r: (u, 0)),
    )
    ys = pl.pallas_call(
        _ffn_body,
        grid_spec=grid_spec,
        out_shape=jax.ShapeDtypeStruct((P, d), jnp.float32),
    )(te, vt, xs, W1, b1, W2, b2)

    out = pl.kernel(
        _make_combine_body(s),
        out_type=jax.ShapeDtypeStruct((b, s, d), jnp.float32),
        mesh=mesh,
        scratch_types=[pltpu.VMEM((s // NW,), jnp.int32),
                       pltpu.VMEM((s // NW, d), jnp.float32),
                       pltpu.SemaphoreType.DMA],
        compiler_params=_SC_PARAMS,
    )(ys, pos)

    return out
